# baseline (speedup 1.0000x reference)
import jax
import jax.numpy as jnp
from jax.experimental import pallas as pl

_HID = 32


def _shift_up(x, s, axis):
    if axis == 0:
        z = jnp.zeros((s, x.shape[1]), x.dtype)
        return jnp.concatenate([x[s:, :], z], axis=0)
    z = jnp.zeros((x.shape[0], s), x.dtype)
    return jnp.concatenate([x[:, s:], z], axis=1)


def _shift_down(x, s, axis):
    if axis == 0:
        z = jnp.zeros((s, x.shape[1]), x.dtype)
        return jnp.concatenate([z, x[:-s, :]], axis=0)
    z = jnp.zeros((x.shape[0], s), x.dtype)
    return jnp.concatenate([z, x[:, :-s]], axis=1)


def _dilate15(x, axis):
    z = _shift_down(x, 7, axis)
    x2 = jnp.maximum(z, _shift_up(z, 1, axis))
    x4 = jnp.maximum(x2, _shift_up(x2, 2, axis))
    x8 = jnp.maximum(x4, _shift_up(x4, 4, axis))
    return jnp.maximum(x8, _shift_up(x8, 7, axis))


def _fused_kernel(img_ref, lr_ref, p_ref, o_ref):
    lr = lr_ref[0, 0]
    one = jnp.float32(1.0)
    zero = jnp.float32(0.0)
    trans = jnp.where((lr > 0.01) & (lr < 0.99), one, zero)
    m = _dilate15(trans, axis=1)
    m = _dilate15(m, axis=0)

    r = img_ref[0, 0]
    g = img_ref[0, 1]
    b = img_ref[0, 2]
    lrn = (lr - 0.5) * 2.0

    p = p_ref[...]
    acc = jnp.full_like(lr, p[6, 0])
    for c in range(_HID):
        h = r * p[0, c] + g * p[1, c] + b * p[2, c] + lrn * p[3, c] + p[4, c]
        h = jnp.maximum(h, zero)
        acc = acc + h * p[5, c]
    pred = jax.nn.sigmoid(acc)
    o_ref[0, 0] = jnp.where(m > 0, pred, lr)


def kernel(image, lr_pred, W1, b1, W2, b2):
    B, _, H, W = image.shape
    params = jnp.zeros((8, _HID), jnp.float32)
    params = params.at[0:4, :].set(W1)
    params = params.at[4, :].set(b1)
    params = params.at[5, :].set(W2[:, 0])
    params = params.at[6, 0].set(b2[0])

    return pl.pallas_call(
        _fused_kernel,
        grid=(B,),
        in_specs=[
            pl.BlockSpec((1, 3, H, W), lambda i: (i, 0, 0, 0)),
            pl.BlockSpec((1, 1, H, W), lambda i: (i, 0, 0, 0)),
            pl.BlockSpec((8, _HID), lambda i: (0, 0)),
        ],
        out_specs=pl.BlockSpec((1, 1, H, W), lambda i: (i, 0, 0, 0)),
        out_shape=jax.ShapeDtypeStruct((B, 1, H, W), jnp.float32),
    )(image, lr_pred, params)


# speedup vs baseline: 1.0001x; 1.0001x over previous
import jax
import jax.numpy as jnp
from jax.experimental import pallas as pl
from jax.experimental.pallas import tpu as pltpu

_HID = 32


def _shift_up(x, s, axis):
    if axis == 0:
        z = jnp.zeros((s, x.shape[1]), x.dtype)
        return jnp.concatenate([x[s:, :], z], axis=0)
    z = jnp.zeros((x.shape[0], s), x.dtype)
    return jnp.concatenate([x[:, s:], z], axis=1)


def _shift_down(x, s, axis):
    if axis == 0:
        z = jnp.zeros((s, x.shape[1]), x.dtype)
        return jnp.concatenate([z, x[:-s, :]], axis=0)
    z = jnp.zeros((x.shape[0], s), x.dtype)
    return jnp.concatenate([z, x[:, :-s]], axis=1)


def _dilate15(x, axis):
    z = _shift_down(x, 7, axis)
    x2 = jnp.maximum(z, _shift_up(z, 1, axis))
    x4 = jnp.maximum(x2, _shift_up(x2, 2, axis))
    x8 = jnp.maximum(x4, _shift_up(x4, 4, axis))
    return jnp.maximum(x8, _shift_up(x8, 7, axis))


def _fused_kernel(img_ref, lr_ref, p_ref, o_ref):
    lr = lr_ref[0, 0]
    one = jnp.float32(1.0)
    zero = jnp.float32(0.0)
    trans = jnp.where((lr > 0.01) & (lr < 0.99), one, zero)
    m = _dilate15(trans, axis=1)
    m = _dilate15(m, axis=0)

    r = img_ref[0, 0]
    g = img_ref[0, 1]
    b = img_ref[0, 2]
    lrn = (lr - 0.5) * 2.0

    p = p_ref[...]
    acc = jnp.full_like(lr, p[6, 0])
    for c in range(_HID):
        h = r * p[0, c] + g * p[1, c] + b * p[2, c] + lrn * p[3, c] + p[4, c]
        h = jnp.maximum(h, zero)
        acc = acc + h * p[5, c]
    pred = jax.nn.sigmoid(acc)
    o_ref[0, 0] = jnp.where(m > 0, pred, lr)


def kernel(image, lr_pred, W1, b1, W2, b2):
    B, _, H, W = image.shape
    params = jnp.zeros((8, _HID), jnp.float32)
    params = params.at[0:4, :].set(W1)
    params = params.at[4, :].set(b1)
    params = params.at[5, :].set(W2[:, 0])
    params = params.at[6, 0].set(b2[0])

    return pl.pallas_call(
        _fused_kernel,
        grid=(B,),
        in_specs=[
            pl.BlockSpec((1, 3, H, W), lambda i: (i, 0, 0, 0)),
            pl.BlockSpec((1, 1, H, W), lambda i: (i, 0, 0, 0)),
            pl.BlockSpec((8, _HID), lambda i: (0, 0)),
        ],
        out_specs=pl.BlockSpec((1, 1, H, W), lambda i: (i, 0, 0, 0)),
        out_shape=jax.ShapeDtypeStruct((B, 1, H, W), jnp.float32),
        compiler_params=pltpu.CompilerParams(
            dimension_semantics=("parallel",),
        ),
    )(image, lr_pred, params)


# speedup vs baseline: 2.3907x; 2.3904x over previous
import jax
import jax.numpy as jnp
from jax.experimental import pallas as pl
from jax.experimental.pallas import tpu as pltpu

_HID = 32


def _shift_up(x, s, axis):
    if axis == 0:
        z = jnp.zeros((s, x.shape[1]), x.dtype)
        return jnp.concatenate([x[s:, :], z], axis=0)
    z = jnp.zeros((x.shape[0], s), x.dtype)
    return jnp.concatenate([x[:, s:], z], axis=1)


def _shift_down(x, s, axis):
    if axis == 0:
        z = jnp.zeros((s, x.shape[1]), x.dtype)
        return jnp.concatenate([z, x[:-s, :]], axis=0)
    z = jnp.zeros((x.shape[0], s), x.dtype)
    return jnp.concatenate([z, x[:, :-s]], axis=1)


def _dilate15(x, axis):
    z = _shift_down(x, 7, axis)
    x2 = jnp.maximum(z, _shift_up(z, 1, axis))
    x4 = jnp.maximum(x2, _shift_up(x2, 2, axis))
    x8 = jnp.maximum(x4, _shift_up(x4, 4, axis))
    return jnp.maximum(x8, _shift_up(x8, 7, axis))


def _fused_kernel(img_ref, lr_ref, p_ref, o_ref):
    lr = lr_ref[0, 0]
    one = jnp.float32(1.0)
    zero = jnp.float32(0.0)
    trans = jnp.where((lr > 0.01) & (lr < 0.99), one, zero)
    m = _dilate15(trans, axis=1)
    m = _dilate15(m, axis=0)

    bf = jnp.bfloat16
    r = img_ref[0, 0].astype(bf)
    g = img_ref[0, 1].astype(bf)
    b = img_ref[0, 2].astype(bf)
    lrn = ((lr - 0.5) * 2.0).astype(bf)

    p = p_ref[...]
    acc = jnp.zeros_like(r) + p[6:7, 0:1]
    zb = bf(0.0)
    for c in range(_HID):
        h = (r * p[0:1, c:c + 1] + g * p[1:2, c:c + 1]
             + b * p[2:3, c:c + 1] + lrn * p[3:4, c:c + 1]
             + p[4:5, c:c + 1])
        h = jnp.maximum(h, zb)
        acc = acc + h * p[5:6, c:c + 1]
    pred = jax.nn.sigmoid(acc.astype(jnp.float32))
    o_ref[0, 0] = jnp.where(m > 0, pred, lr)


def kernel(image, lr_pred, W1, b1, W2, b2):
    B, _, H, W = image.shape
    params = jnp.zeros((8, _HID), jnp.float32)
    params = params.at[0:4, :].set(W1)
    params = params.at[4, :].set(b1)
    params = params.at[5, :].set(W2[:, 0])
    params = params.at[6, 0].set(b2[0])
    params = params.astype(jnp.bfloat16)

    return pl.pallas_call(
        _fused_kernel,
        grid=(B,),
        in_specs=[
            pl.BlockSpec((1, 3, H, W), lambda i: (i, 0, 0, 0)),
            pl.BlockSpec((1, 1, H, W), lambda i: (i, 0, 0, 0)),
            pl.BlockSpec((8, _HID), lambda i: (0, 0)),
        ],
        out_specs=pl.BlockSpec((1, 1, H, W), lambda i: (i, 0, 0, 0)),
        out_shape=jax.ShapeDtypeStruct((B, 1, H, W), jnp.float32),
        compiler_params=pltpu.CompilerParams(
            dimension_semantics=("parallel",),
        ),
    )(image, lr_pred, params)


# speedup vs baseline: 2.4076x; 1.0071x over previous
import jax
import jax.numpy as jnp
from jax.experimental import pallas as pl
from jax.experimental.pallas import tpu as pltpu

_HID = 32


def _shift_up(x, s, axis):
    if axis == 0:
        z = jnp.zeros((s, x.shape[1]), x.dtype)
        return jnp.concatenate([x[s:, :], z], axis=0)
    z = jnp.zeros((x.shape[0], s), x.dtype)
    return jnp.concatenate([x[:, s:], z], axis=1)


def _shift_down(x, s, axis):
    if axis == 0:
        z = jnp.zeros((s, x.shape[1]), x.dtype)
        return jnp.concatenate([z, x[:-s, :]], axis=0)
    z = jnp.zeros((x.shape[0], s), x.dtype)
    return jnp.concatenate([z, x[:, :-s]], axis=1)


def _dilate15(x, axis):
    z = _shift_down(x, 7, axis)
    x2 = jnp.maximum(z, _shift_up(z, 1, axis))
    x4 = jnp.maximum(x2, _shift_up(x2, 2, axis))
    x8 = jnp.maximum(x4, _shift_up(x4, 4, axis))
    return jnp.maximum(x8, _shift_up(x8, 7, axis))


def _fused_kernel(img_ref, lr_ref, p_ref, o_ref):
    bf = jnp.bfloat16
    lr = lr_ref[0, 0]
    trans = jnp.where((lr > 0.01) & (lr < 0.99),
                      jnp.float32(1.0), jnp.float32(0.0)).astype(bf)
    m = _dilate15(trans, axis=1)
    m = _dilate15(m, axis=0).astype(jnp.float32)
    r = img_ref[0, 0].astype(bf)
    g = img_ref[0, 1].astype(bf)
    b = img_ref[0, 2].astype(bf)
    lrn = ((lr - 0.5) * 2.0).astype(bf)

    p = p_ref[...]
    acc = jnp.zeros_like(r) + p[6:7, 0:1]
    zb = bf(0.0)
    for c in range(_HID):
        h = (r * p[0:1, c:c + 1] + g * p[1:2, c:c + 1]
             + b * p[2:3, c:c + 1] + lrn * p[3:4, c:c + 1]
             + p[4:5, c:c + 1])
        h = jnp.maximum(h, zb)
        acc = acc + h * p[5:6, c:c + 1]
    pred = jax.nn.sigmoid(acc)
    o_ref[0, 0] = jnp.where(m > 0, pred.astype(jnp.float32), lr)


def kernel(image, lr_pred, W1, b1, W2, b2):
    B, _, H, W = image.shape
    params = jnp.zeros((8, _HID), jnp.float32)
    params = params.at[0:4, :].set(W1)
    params = params.at[4, :].set(b1)
    params = params.at[5, :].set(W2[:, 0])
    params = params.at[6, 0].set(b2[0])
    params = params.astype(jnp.bfloat16)

    return pl.pallas_call(
        _fused_kernel,
        grid=(B,),
        in_specs=[
            pl.BlockSpec((1, 3, H, W), lambda i: (i, 0, 0, 0)),
            pl.BlockSpec((1, 1, H, W), lambda i: (i, 0, 0, 0)),
            pl.BlockSpec((8, _HID), lambda i: (0, 0)),
        ],
        out_specs=pl.BlockSpec((1, 1, H, W), lambda i: (i, 0, 0, 0)),
        out_shape=jax.ShapeDtypeStruct((B, 1, H, W), jnp.float32),
        compiler_params=pltpu.CompilerParams(
            dimension_semantics=("parallel",),
        ),
    )(image, lr_pred, params)


# speedup vs baseline: 2.4110x; 1.0014x over previous
import jax
import jax.numpy as jnp
from jax.experimental import pallas as pl
from jax.experimental.pallas import tpu as pltpu

_HID = 32


def _shift_up(x, s, axis):
    if axis == 0:
        z = jnp.zeros((s, x.shape[1]), x.dtype)
        return jnp.concatenate([x[s:, :], z], axis=0)
    z = jnp.zeros((x.shape[0], s), x.dtype)
    return jnp.concatenate([x[:, s:], z], axis=1)


def _shift_down(x, s, axis):
    if axis == 0:
        z = jnp.zeros((s, x.shape[1]), x.dtype)
        return jnp.concatenate([z, x[:-s, :]], axis=0)
    z = jnp.zeros((x.shape[0], s), x.dtype)
    return jnp.concatenate([z, x[:, :-s]], axis=1)


def _dilate15(x, axis):
    z = _shift_down(x, 7, axis)
    x2 = jnp.maximum(z, _shift_up(z, 1, axis))
    x4 = jnp.maximum(x2, _shift_up(x2, 2, axis))
    x8 = jnp.maximum(x4, _shift_up(x4, 4, axis))
    return jnp.maximum(x8, _shift_up(x8, 7, axis))


_TILE = 64


def _fused_kernel(img_ref, lr_ref, p_ref, o_ref):
    bf = jnp.bfloat16
    lr = lr_ref[0, 0]
    trans = jnp.where((lr > 0.01) & (lr < 0.99),
                      jnp.float32(1.0), jnp.float32(0.0)).astype(bf)
    m = _dilate15(trans, axis=1)
    m = _dilate15(m, axis=0)

    p = p_ref[...]
    zb = bf(0.0)
    H = lr.shape[0]
    for t in range(H // _TILE):
        sl = slice(t * _TILE, (t + 1) * _TILE)
        lrt = lr_ref[0, 0, sl, :]
        r = img_ref[0, 0, sl, :].astype(bf)
        g = img_ref[0, 1, sl, :].astype(bf)
        b = img_ref[0, 2, sl, :].astype(bf)
        lrn = ((lrt - 0.5) * 2.0).astype(bf)
        acc = jnp.zeros_like(r) + p[6:7, 0:1]
        for c in range(_HID):
            h = (r * p[0:1, c:c + 1] + g * p[1:2, c:c + 1]
                 + b * p[2:3, c:c + 1] + lrn * p[3:4, c:c + 1]
                 + p[4:5, c:c + 1])
            h = jnp.maximum(h, zb)
            acc = acc + h * p[5:6, c:c + 1]
        pred = jax.nn.sigmoid(acc)
        mt = m[sl, :].astype(jnp.float32)
        o_ref[0, 0, sl, :] = jnp.where(mt > 0, pred.astype(jnp.float32), lrt)


def kernel(image, lr_pred, W1, b1, W2, b2):
    B, _, H, W = image.shape
    params = jnp.zeros((8, _HID), jnp.float32)
    params = params.at[0:4, :].set(W1)
    params = params.at[4, :].set(b1)
    params = params.at[5, :].set(W2[:, 0])
    params = params.at[6, 0].set(b2[0])
    params = params.astype(jnp.bfloat16)

    return pl.pallas_call(
        _fused_kernel,
        grid=(B,),
        in_specs=[
            pl.BlockSpec((1, 3, H, W), lambda i: (i, 0, 0, 0)),
            pl.BlockSpec((1, 1, H, W), lambda i: (i, 0, 0, 0)),
            pl.BlockSpec((8, _HID), lambda i: (0, 0)),
        ],
        out_specs=pl.BlockSpec((1, 1, H, W), lambda i: (i, 0, 0, 0)),
        out_shape=jax.ShapeDtypeStruct((B, 1, H, W), jnp.float32),
        compiler_params=pltpu.CompilerParams(
            dimension_semantics=("parallel",),
        ),
    )(image, lr_pred, params)


# speedup vs baseline: 2.4511x; 1.0166x over previous
import jax
import jax.numpy as jnp
from jax.experimental import pallas as pl
from jax.experimental.pallas import tpu as pltpu

_HID = 32


def _shift_up(x, s, axis):
    if axis == 0:
        z = jnp.zeros((s, x.shape[1]), x.dtype)
        return jnp.concatenate([x[s:, :], z], axis=0)
    z = jnp.zeros((x.shape[0], s), x.dtype)
    return jnp.concatenate([x[:, s:], z], axis=1)


def _shift_down(x, s, axis):
    if axis == 0:
        z = jnp.zeros((s, x.shape[1]), x.dtype)
        return jnp.concatenate([z, x[:-s, :]], axis=0)
    z = jnp.zeros((x.shape[0], s), x.dtype)
    return jnp.concatenate([z, x[:, :-s]], axis=1)


def _dilate15(x, axis):
    z = _shift_down(x, 7, axis)
    x2 = jnp.maximum(z, _shift_up(z, 1, axis))
    x4 = jnp.maximum(x2, _shift_up(x2, 2, axis))
    x8 = jnp.maximum(x4, _shift_up(x4, 4, axis))
    return jnp.maximum(x8, _shift_up(x8, 7, axis))


_TILE = 64


def _fused_kernel(img_ref, lr_ref, p_ref, o_ref):
    bf = jnp.bfloat16
    lr = lr_ref[0, 0]
    trans = jnp.where((lr > 0.01) & (lr < 0.99),
                      jnp.float32(1.0), jnp.float32(0.0)).astype(bf)
    m = _dilate15(trans, axis=1)
    m = _dilate15(m, axis=0)

    p = p_ref[...]
    zb = bf(0.0)
    H = lr.shape[0]
    for t in range(H // _TILE):
        sl = slice(t * _TILE, (t + 1) * _TILE)
        lrt = lr_ref[0, 0, sl, :]
        r = img_ref[0, 0, sl, :].astype(bf)
        g = img_ref[0, 1, sl, :].astype(bf)
        b = img_ref[0, 2, sl, :].astype(bf)
        lrn = ((lrt - 0.5) * 2.0).astype(bf)
        acc = jnp.zeros_like(r) + p[7:8, 0:1]
        for c in range(_HID):
            h = (r * p[0:1, c:c + 1] + g * p[1:2, c:c + 1]
                 + b * p[2:3, c:c + 1] + lrn * p[3:4, c:c + 1]
                 + p[4:5, c:c + 1])
            h = jnp.minimum(jnp.maximum(h, p[5:6, c:c + 1]), p[6:7, c:c + 1])
            acc = acc + h
        pred = jax.nn.sigmoid(acc)
        mt = m[sl, :].astype(jnp.float32)
        o_ref[0, 0, sl, :] = jnp.where(mt > 0, pred.astype(jnp.float32), lrt)


def kernel(image, lr_pred, W1, b1, W2, b2):
    B, _, H, W = image.shape
    w2 = W2[:, 0]
    params = jnp.zeros((8, _HID), jnp.float32)
    params = params.at[0:4, :].set(W1 * w2[None, :])
    params = params.at[4, :].set(b1 * w2)
    params = params.at[5, :].set(jnp.where(w2 >= 0, 0.0, -jnp.inf))
    params = params.at[6, :].set(jnp.where(w2 >= 0, jnp.inf, 0.0))
    params = params.at[7, 0].set(b2[0])
    params = params.astype(jnp.bfloat16)

    return pl.pallas_call(
        _fused_kernel,
        grid=(B,),
        in_specs=[
            pl.BlockSpec((1, 3, H, W), lambda i: (i, 0, 0, 0)),
            pl.BlockSpec((1, 1, H, W), lambda i: (i, 0, 0, 0)),
            pl.BlockSpec((8, _HID), lambda i: (0, 0)),
        ],
        out_specs=pl.BlockSpec((1, 1, H, W), lambda i: (i, 0, 0, 0)),
        out_shape=jax.ShapeDtypeStruct((B, 1, H, W), jnp.float32),
        compiler_params=pltpu.CompilerParams(
            dimension_semantics=("parallel",),
        ),
    )(image, lr_pred, params)


# speedup vs baseline: 2.5859x; 1.0550x over previous
import jax
import jax.numpy as jnp
from jax.experimental import pallas as pl
from jax.experimental.pallas import tpu as pltpu

_HID = 32
_ROWS = 64


def _shift_up(x, s, axis):
    if axis == 0:
        z = jnp.zeros((s, x.shape[1]), x.dtype)
        return jnp.concatenate([x[s:, :], z], axis=0)
    z = jnp.zeros((x.shape[0], s), x.dtype)
    return jnp.concatenate([x[:, s:], z], axis=1)


def _shift_down(x, s, axis):
    if axis == 0:
        z = jnp.zeros((s, x.shape[1]), x.dtype)
        return jnp.concatenate([z, x[:-s, :]], axis=0)
    z = jnp.zeros((x.shape[0], s), x.dtype)
    return jnp.concatenate([z, x[:, :-s]], axis=1)


def _dilate15(x, axis):
    z = _shift_down(x, 7, axis)
    x2 = jnp.maximum(z, _shift_up(z, 1, axis))
    x4 = jnp.maximum(x2, _shift_up(x2, 2, axis))
    x8 = jnp.maximum(x4, _shift_up(x4, 4, axis))
    return jnp.maximum(x8, _shift_up(x8, 7, axis))


def _mask_kernel(lr_ref, o_ref):
    lr = lr_ref[0, 0]
    trans = jnp.where((lr > 0.01) & (lr < 0.99),
                      jnp.float32(1.0), jnp.float32(0.0)).astype(jnp.bfloat16)
    m = _dilate15(trans, axis=1)
    o_ref[0, 0] = _dilate15(m, axis=0)


def _mlp_kernel(img_ref, lr_ref, m_ref, p_ref, o_ref):
    bf = jnp.bfloat16
    lrt = lr_ref[0, 0]
    r = img_ref[0, 0].astype(bf)
    g = img_ref[0, 1].astype(bf)
    b = img_ref[0, 2].astype(bf)
    lrn = ((lrt - 0.5) * 2.0).astype(bf)

    p = p_ref[...]
    acc = jnp.zeros_like(r) + p[7:8, 0:1]
    for c in range(_HID):
        h = (r * p[0:1, c:c + 1] + g * p[1:2, c:c + 1]
             + b * p[2:3, c:c + 1] + lrn * p[3:4, c:c + 1]
             + p[4:5, c:c + 1])
        h = jnp.minimum(jnp.maximum(h, p[5:6, c:c + 1]), p[6:7, c:c + 1])
        acc = acc + h
    pred = jax.nn.sigmoid(acc)
    mt = m_ref[0, 0].astype(jnp.float32)
    o_ref[0, 0] = jnp.where(mt > 0, pred.astype(jnp.float32), lrt)


def kernel(image, lr_pred, W1, b1, W2, b2):
    B, _, H, W = image.shape
    w2 = W2[:, 0]
    params = jnp.zeros((8, _HID), jnp.float32)
    params = params.at[0:4, :].set(W1 * w2[None, :])
    params = params.at[4, :].set(b1 * w2)
    params = params.at[5, :].set(jnp.where(w2 >= 0, 0.0, -jnp.inf))
    params = params.at[6, :].set(jnp.where(w2 >= 0, jnp.inf, 0.0))
    params = params.at[7, 0].set(b2[0])
    params = params.astype(jnp.bfloat16)

    mask = pl.pallas_call(
        _mask_kernel,
        grid=(B,),
        in_specs=[pl.BlockSpec((1, 1, H, W), lambda i: (i, 0, 0, 0))],
        out_specs=pl.BlockSpec((1, 1, H, W), lambda i: (i, 0, 0, 0)),
        out_shape=jax.ShapeDtypeStruct((B, 1, H, W), jnp.bfloat16),
        compiler_params=pltpu.CompilerParams(
            dimension_semantics=("parallel",),
        ),
    )(lr_pred)

    return pl.pallas_call(
        _mlp_kernel,
        grid=(B, H // _ROWS),
        in_specs=[
            pl.BlockSpec((1, 3, _ROWS, W), lambda i, t: (i, 0, t, 0)),
            pl.BlockSpec((1, 1, _ROWS, W), lambda i, t: (i, 0, t, 0)),
            pl.BlockSpec((1, 1, _ROWS, W), lambda i, t: (i, 0, t, 0)),
            pl.BlockSpec((8, _HID), lambda i, t: (0, 0)),
        ],
        out_specs=pl.BlockSpec((1, 1, _ROWS, W), lambda i, t: (i, 0, t, 0)),
        out_shape=jax.ShapeDtypeStruct((B, 1, H, W), jnp.float32),
        compiler_params=pltpu.CompilerParams(
            dimension_semantics=("parallel", "arbitrary"),
        ),
    )(image, lr_pred, mask, params)
